# baseline (speedup 1.0000x reference)
import jax
import jax.numpy as jnp
from jax import lax
from jax.experimental import pallas as pl
from jax.experimental.pallas import tpu as pltpu
from jax.experimental.pallas import tpu_sc as plsc

N = 50000
E = 800000
DIM = 2
SEQ = 5
LATENT = 64
H = 0.05
CLAMP = 1.0
VEL_W = 1.0
STD = 1.0
MEAN = 0.0
B_LO = 0.0
B_HI = 1.0

NC = 2
NS = 16
NW = NC * NS
SUB = 80
NSUB = 8
CH = SUB * NSUB

HALF = N // 2
PROWS = 12544
TRASH = PROWS - 1

BN = 5000
BE = 4000

_f32 = jnp.float32


def _ln(h, g, b):
    m = jnp.mean(h, axis=-1, keepdims=True)
    v = jnp.mean((h - m) * (h - m), axis=-1, keepdims=True)
    return (h - m) * lax.rsqrt(v + 1e-5) * g + b


def _sc_gather(table, idx3d, K, outw):
    nch = K // CH
    niter = -(-nch // NW)
    mesh = plsc.VectorSubcoreMesh(core_axis_name="c", subcore_axis_name="s")

    def body(table_hbm, idx_hbm, out_hbm, idx_v, rows_v, sem, semw):
        cid = lax.axis_index("c")
        sid = lax.axis_index("s")
        wid = sid * NC + cid

        def _drain_put(ch, j):
            pltpu.make_async_copy(
                rows_v.at[pl.ds(j * SUB, SUB)],
                out_hbm.at[pl.ds(ch * CH + j * SUB, SUB)],
                semw,
            ).wait()

        def it(i, carry):
            ch = i * NW + wid

            @pl.when(ch < nch)
            def _do():
                pltpu.sync_copy(idx_hbm.at[ch], idx_v)
                gets = []
                for j in range(NSUB):
                    @pl.when(i > 0)
                    def _(_j=j):
                        _drain_put(ch, _j)

                    gets.append(pltpu.async_copy(
                        table_hbm.at[idx_v.at[j]],
                        rows_v.at[pl.ds(j * SUB, SUB)],
                        sem,
                    ))
                for j in range(NSUB):
                    gets[j].wait()
                    pltpu.async_copy(
                        rows_v.at[pl.ds(j * SUB, SUB)],
                        out_hbm.at[pl.ds(ch * CH + j * SUB, SUB)],
                        semw,
                    )

            return carry

        lax.fori_loop(0, niter, it, 0)
        for j in range(NSUB):
            _drain_put(0, j)

    run = pl.kernel(
        body,
        out_type=jax.ShapeDtypeStruct((K, 128), _f32),
        mesh=mesh,
        scratch_types=[
            pltpu.VMEM((NSUB, SUB), jnp.int32),
            pltpu.VMEM((CH, 128), _f32),
            pltpu.SemaphoreType.DMA,
            pltpu.SemaphoreType.DMA,
        ],
    )
    return run(table, idx3d)


def _sc_scatter_add(eupd_sl, dst3d, ne):
    TS = PROWS // NS
    nch = ne // CH
    niter = -(-nch // NS)
    mesh = plsc.VectorSubcoreMesh(core_axis_name="c", subcore_axis_name="s")

    def body(eupd_hbm, dst_hbm, zeros_hbm, out_hbm,
             dst_a, dst_b, lidx_v, rows_a, rows_b, zero_v, shared,
             seml, sema, semd):
        cid = lax.axis_index("c")
        sid = lax.axis_index("s")
        node0 = cid * HALF
        bufs = (rows_a, rows_b)
        dbufs = (dst_a, dst_b)

        pltpu.sync_copy(zeros_hbm, zero_v)
        zcopies = [
            pltpu.async_copy(zero_v if k < 16 else zero_v.at[pl.ds(0, 16)],
                             shared.at[pl.ds(sid * TS + k * 48,
                                             48 if k < 16 else 16)],
                             sema)
            for k in range(17)
        ]
        for z in zcopies:
            z.wait()
        plsc.subcore_barrier()

        def chunk(ch, dbuf, nbuf, next_ch):
            @pl.when(next_ch < nch)
            def _():
                pltpu.async_copy(dst_hbm.at[next_ch], nbuf, semd)

            for j in range(NSUB):
                for k in range(SUB // 16):
                    d = dbuf[j, pl.ds(k * 16, 16)]
                    li = d - node0
                    ok = (li >= 0) & (li < HALF)
                    r = lax.shift_right_logical(li, 1)
                    lidx_v[j, pl.ds(k * 16, 16)] = jnp.where(ok, r, TRASH)
            loads = [None] * NSUB
            adds = [None] * NSUB
            loads[0] = pltpu.async_copy(
                eupd_hbm.at[pl.ds(ch * CH, SUB)], bufs[0], seml)
            for sb in range(NSUB):
                pr = sb & 1
                loads[sb].wait()
                adds[sb] = pltpu.async_copy(
                    bufs[pr], shared.at[lidx_v.at[sb]], sema, add=True)
                if sb + 1 < NSUB:
                    if sb >= 1:
                        adds[sb - 1].wait()
                    loads[sb + 1] = pltpu.async_copy(
                        eupd_hbm.at[pl.ds(ch * CH + (sb + 1) * SUB, SUB)],
                        bufs[1 - pr], seml)
            adds[NSUB - 2].wait()
            adds[NSUB - 1].wait()

        def _drain_dst(nbuf, next_ch):
            pltpu.make_async_copy(dst_hbm.at[next_ch], nbuf, semd).wait()

        pltpu.sync_copy(dst_hbm.at[sid], dst_a)

        def it(m, carry):
            ch0 = (2 * m) * NS + sid
            ch1 = ch0 + NS
            ch2 = ch0 + 2 * NS

            @pl.when(ch0 < nch)
            def _c0():
                chunk(ch0, dst_a, dst_b, ch1)

            @pl.when(ch1 < nch)
            def _c1():
                _drain_dst(dst_b, ch1)
                chunk(ch1, dst_b, dst_a, ch2)

            @pl.when(ch2 < nch)
            def _c2():
                _drain_dst(dst_a, ch2)

            return carry

        lax.fori_loop(0, -(-niter // 2), it, 0)
        plsc.subcore_barrier()

        puts = []
        for k in range(10):
            r0 = sid * TS + k * 80
            nrow = 80 if k < 9 else 784 - 720
            buf = bufs[k & 1]
            if k >= 2:
                puts[k - 2].wait()
            pltpu.sync_copy(shared.at[pl.ds(r0, nrow)],
                            buf.at[pl.ds(0, nrow)])
            puts.append(pltpu.async_copy(
                buf.at[pl.ds(0, nrow)],
                out_hbm.at[pl.ds(cid * PROWS + r0, nrow)], seml))
        puts[-2].wait()
        puts[-1].wait()

    run = pl.kernel(
        body,
        out_type=jax.ShapeDtypeStruct((2 * PROWS, 128), _f32),
        mesh=mesh,
        scratch_types=[
            pltpu.VMEM((NSUB, SUB), jnp.int32),
            pltpu.VMEM((NSUB, SUB), jnp.int32),
            pltpu.VMEM((NSUB, SUB), jnp.int32),
            pltpu.VMEM((SUB, 128), _f32),
            pltpu.VMEM((SUB, 128), _f32),
            pltpu.VMEM((48, 128), _f32),
            pltpu.VMEM_SHARED((PROWS, 128), _f32),
            pltpu.SemaphoreType.DMA,
            pltpu.SemaphoreType.DMA,
            pltpu.SemaphoreType.DMA,
        ],
    )
    zeros = jnp.zeros((48, 128), _f32)
    return run(eupd_sl, dst3d, zeros)


def _full(shape):
    return pl.BlockSpec(shape, lambda i: (0, 0))


_AGG_SPEC = pl.BlockSpec((1, BN, LATENT), lambda i: (i // 5, i % 5, 0))


def _node_encode(pos, vel, w1v, w1dl, w1du, b1, w2, b2, g, b):
    def body(pos_ref, vel_ref, w1v_r, w1dl_r, w1du_r, b1_r, w2_r, b2_r,
             g_r, b_r, x_ref, xp_ref):
        p = pos_ref[...]
        v = vel_ref[...] * VEL_W
        dl = jnp.clip(jnp.abs(p - B_LO) / H, -CLAMP, CLAMP)
        du = jnp.clip(jnp.abs(B_HI - p) / H, -CLAMP, CLAMP)
        h = jnp.maximum(
            jnp.dot(v, w1v_r[...], preferred_element_type=_f32, precision=jax.lax.Precision.HIGHEST)
            + jnp.dot(dl, w1dl_r[...], preferred_element_type=_f32, precision=jax.lax.Precision.HIGHEST)
            + jnp.dot(du, w1du_r[...], preferred_element_type=_f32, precision=jax.lax.Precision.HIGHEST)
            + b1_r[...], 0.0)
        h2 = jnp.dot(h, w2_r[...], preferred_element_type=_f32, precision=jax.lax.Precision.HIGHEST) + b2_r[...]
        x = _ln(h2, g_r[...], b_r[...])
        x_ref[...] = x
        xp_ref[...] = jnp.concatenate(
            [x, p, jnp.zeros((p.shape[0], 128 - LATENT - DIM), _f32)], axis=1)

    return pl.pallas_call(
        body,
        grid=(N // BN,),
        in_specs=[
            pl.BlockSpec((BN, DIM), lambda i: (i, 0)),
            pl.BlockSpec((BN, SEQ * DIM), lambda i: (i, 0)),
            _full((SEQ * DIM, LATENT)), _full((DIM, LATENT)),
            _full((DIM, LATENT)), _full((1, LATENT)),
            _full((LATENT, LATENT)), _full((1, LATENT)),
            _full((1, LATENT)), _full((1, LATENT)),
        ],
        out_specs=[
            pl.BlockSpec((BN, LATENT), lambda i: (i, 0)),
            pl.BlockSpec((BN, 128), lambda i: (i, 0)),
        ],
        out_shape=[
            jax.ShapeDtypeStruct((N, LATENT), _f32),
            jax.ShapeDtypeStruct((N, 128), _f32),
        ],
    )(pos, vel, w1v, w1dl, w1du, b1, w2, b2, g, b)


def _edge_step1(G1, pdst, wr, wd, be1, we2, be2, ge, bel,
                wse, wss, wsr, bs1, ws2, bs2, gs, bsl):
    def body(s_ref, r_ref, p_ref, wr_r, wd_r, be1_r, we2_r, be2_r, ge_r,
             bel_r, wse_r, wss_r, wsr_r, bs1_r, ws2_r, bs2_r, gs_r, bsl_r,
             e1_ref, eu_ref):
        s = s_ref[...]
        r = r_ref[...]
        sx = s[:, :LATENT]
        rx = r[:, :LATENT]
        rel = (s[:, LATENT:LATENT + DIM] - r[:, LATENT:LATENT + DIM]) / H
        rd = jnp.sqrt(jnp.sum(rel * rel, axis=-1, keepdims=True))
        he = jnp.maximum(
            jnp.dot(rel, wr_r[...], preferred_element_type=_f32, precision=jax.lax.Precision.HIGHEST)
            + rd * wd_r[...] + be1_r[...], 0.0)
        e = _ln(jnp.dot(he, we2_r[...], preferred_element_type=_f32, precision=jax.lax.Precision.HIGHEST)
                + be2_r[...], ge_r[...], bel_r[...])
        h = jnp.maximum(
            jnp.dot(e, wse_r[...], preferred_element_type=_f32, precision=jax.lax.Precision.HIGHEST)
            + jnp.dot(sx, wss_r[...], preferred_element_type=_f32, precision=jax.lax.Precision.HIGHEST)
            + jnp.dot(rx, wsr_r[...], preferred_element_type=_f32, precision=jax.lax.Precision.HIGHEST)
            + bs1_r[...], 0.0)
        eu = _ln(jnp.dot(h, ws2_r[...], preferred_element_type=_f32, precision=jax.lax.Precision.HIGHEST)
                 + bs2_r[...], gs_r[...], bsl_r[...])
        e1_ref[...] = e + eu
        p = p_ref[...]
        eu_ref[...] = jnp.concatenate([eu * (1.0 - p), eu * p], axis=1)

    ne = pdst.shape[0]
    nb = ne // BE
    return pl.pallas_call(
        body,
        grid=(nb,),
        in_specs=[
            pl.BlockSpec((BE, 128), lambda i: (i, 0)),
            pl.BlockSpec((BE, 128), lambda i, _nb=nb: (_nb + i, 0)),
            pl.BlockSpec((BE, 1), lambda i: (i, 0)),
            _full((DIM, LATENT)), _full((1, LATENT)), _full((1, LATENT)),
            _full((LATENT, LATENT)), _full((1, LATENT)),
            _full((1, LATENT)), _full((1, LATENT)),
            _full((LATENT, LATENT)), _full((LATENT, LATENT)),
            _full((LATENT, LATENT)), _full((1, LATENT)),
            _full((LATENT, LATENT)), _full((1, LATENT)),
            _full((1, LATENT)), _full((1, LATENT)),
        ],
        out_specs=[
            pl.BlockSpec((BE, LATENT), lambda i: (i, 0)),
            pl.BlockSpec((BE, 128), lambda i: (i, 0)),
        ],
        out_shape=[
            jax.ShapeDtypeStruct((ne, LATENT), _f32),
            jax.ShapeDtypeStruct((ne, 128), _f32),
        ],
    )(G1, G1, pdst, wr, wd, be1, we2, be2, ge, bel,
      wse, wss, wsr, bs1, ws2, bs2, gs, bsl)


def _edge_step2(e1, G2, pdst, wse, wss, wsr, bs1, ws2, bs2, gs, bsl):
    def body(e_ref, s_ref, r_ref, p_ref, wse_r, wss_r, wsr_r, bs1_r, ws2_r,
             bs2_r, gs_r, bsl_r, eu_ref):
        h = jnp.maximum(
            jnp.dot(e_ref[...], wse_r[...], preferred_element_type=_f32, precision=jax.lax.Precision.HIGHEST)
            + jnp.dot(s_ref[...][:, :LATENT], wss_r[...],
                      preferred_element_type=_f32, precision=jax.lax.Precision.HIGHEST)
            + jnp.dot(r_ref[...][:, :LATENT], wsr_r[...],
                      preferred_element_type=_f32, precision=jax.lax.Precision.HIGHEST)
            + bs1_r[...], 0.0)
        eu = _ln(jnp.dot(h, ws2_r[...], preferred_element_type=_f32, precision=jax.lax.Precision.HIGHEST)
                 + bs2_r[...], gs_r[...], bsl_r[...])
        p = p_ref[...]
        eu_ref[...] = jnp.concatenate([eu * (1.0 - p), eu * p], axis=1)

    ne = pdst.shape[0]
    nb = ne // BE
    return pl.pallas_call(
        body,
        grid=(nb,),
        in_specs=[
            pl.BlockSpec((BE, LATENT), lambda i: (i, 0)),
            pl.BlockSpec((BE, 128), lambda i: (i, 0)),
            pl.BlockSpec((BE, 128), lambda i, _nb=nb: (_nb + i, 0)),
            pl.BlockSpec((BE, 1), lambda i: (i, 0)),
            _full((LATENT, LATENT)), _full((LATENT, LATENT)),
            _full((LATENT, LATENT)), _full((1, LATENT)),
            _full((LATENT, LATENT)), _full((1, LATENT)),
            _full((1, LATENT)), _full((1, LATENT)),
        ],
        out_specs=pl.BlockSpec((BE, 128), lambda i: (i, 0)),
        out_shape=jax.ShapeDtypeStruct((ne, 128), _f32),
    )(e1, G2, G2, pdst, wse, wss, wsr, bs1, ws2, bs2, gs, bsl)


def _node_update(x, agg, wnx, wna, bn1, wn2, bn2, gn, bnl):
    def body(x_ref, a_ref, wnx_r, wna_r, bn1_r, wn2_r, bn2_r, gn_r, bnl_r,
             o_ref, ot_ref):
        x_ = x_ref[...]
        h = jnp.maximum(
            jnp.dot(x_, wnx_r[...], preferred_element_type=_f32, precision=jax.lax.Precision.HIGHEST)
            + jnp.dot(a_ref[0], wna_r[...], preferred_element_type=_f32, precision=jax.lax.Precision.HIGHEST)
            + bn1_r[...], 0.0)
        x1 = x_ + _ln(
            jnp.dot(h, wn2_r[...], preferred_element_type=_f32, precision=jax.lax.Precision.HIGHEST) + bn2_r[...],
            gn_r[...], bnl_r[...])
        o_ref[...] = x1
        ot_ref[...] = jnp.concatenate(
            [x1, jnp.zeros((x1.shape[0], 128 - LATENT), _f32)], axis=1)

    return pl.pallas_call(
        body,
        grid=(N // BN,),
        in_specs=[
            pl.BlockSpec((BN, LATENT), lambda i: (i, 0)),
            _AGG_SPEC,
            _full((LATENT, LATENT)), _full((LATENT, LATENT)),
            _full((1, LATENT)), _full((LATENT, LATENT)), _full((1, LATENT)),
            _full((1, LATENT)), _full((1, LATENT)),
        ],
        out_specs=[
            pl.BlockSpec((BN, LATENT), lambda i: (i, 0)),
            pl.BlockSpec((BN, 128), lambda i: (i, 0)),
        ],
        out_shape=[
            jax.ShapeDtypeStruct((N, LATENT), _f32),
            jax.ShapeDtypeStruct((N, 128), _f32),
        ],
    )(x, agg, wnx, wna, bn1, wn2, bn2, gn, bnl)


def _node_update_decode(x, agg, wnx, wna, bn1, wn2, bn2, gn, bnl,
                        wd1, bd1, wd2, bd2):
    def body(x_ref, a_ref, wnx_r, wna_r, bn1_r, wn2_r, bn2_r, gn_r, bnl_r,
             wd1_r, bd1_r, wd2_r, bd2_r, o_ref):
        x_ = x_ref[...]
        h = jnp.maximum(
            jnp.dot(x_, wnx_r[...], preferred_element_type=_f32, precision=jax.lax.Precision.HIGHEST)
            + jnp.dot(a_ref[0], wna_r[...], preferred_element_type=_f32, precision=jax.lax.Precision.HIGHEST)
            + bn1_r[...], 0.0)
        x2 = x_ + _ln(
            jnp.dot(h, wn2_r[...], preferred_element_type=_f32, precision=jax.lax.Precision.HIGHEST) + bn2_r[...],
            gn_r[...], bnl_r[...])
        hd = jnp.maximum(
            jnp.dot(x2, wd1_r[...], preferred_element_type=_f32, precision=jax.lax.Precision.HIGHEST) + bd1_r[...],
            0.0)
        o_ref[...] = (jnp.dot(hd, wd2_r[...], preferred_element_type=_f32, precision=jax.lax.Precision.HIGHEST)
                      + bd2_r[...]) * STD + MEAN

    return pl.pallas_call(
        body,
        grid=(N // BN,),
        in_specs=[
            pl.BlockSpec((BN, LATENT), lambda i: (i, 0)),
            _AGG_SPEC,
            _full((LATENT, LATENT)), _full((LATENT, LATENT)),
            _full((1, LATENT)), _full((LATENT, LATENT)), _full((1, LATENT)),
            _full((1, LATENT)), _full((1, LATENT)),
            _full((LATENT, LATENT)), _full((1, LATENT)),
            _full((LATENT, DIM)), _full((1, DIM)),
        ],
        out_specs=pl.BlockSpec((BN, DIM), lambda i: (i, 0)),
        out_shape=jax.ShapeDtypeStruct((N, DIM), _f32),
    )(x, agg, wnx, wna, bn1, wn2, bn2, gn, bnl, wd1, bd1, wd2, bd2)


def _row(v):
    return v.reshape(1, -1)


def kernel(current_position, velocity_sequence, edge_index, params):
    pos = current_position.astype(_f32)
    vel = velocity_sequence.astype(_f32).reshape(N, SEQ * DIM)
    ei = edge_index.astype(jnp.int32)
    PARTS = (272000, 272000, 256000)
    halves = []
    off = 0
    for ne in PARTS:
        eh = lax.slice(ei, (0, off), (2, off + ne))
        off += ne
        halves.append((
            eh.reshape(2 * ne // CH, NSUB, SUB),
            eh[1].reshape(ne // CH, NSUB, SUB),
            (eh[1] & 1).astype(_f32).reshape(ne, 1),
            ne,
        ))

    (wn1, bn1e), (wn2e, bn2e) = params["node_enc"]
    gn_e, bn_e = params["node_enc_ln"]
    (we1, be1), (we2, be2) = params["edge_enc"]
    ge_e, be_l = params["edge_enc_ln"]
    (wd1, bd1), (wd2, bd2) = params["dec"]

    x0, xp0 = _node_encode(
        pos, vel,
        wn1[:SEQ * DIM], wn1[SEQ * DIM:SEQ * DIM + DIM],
        wn1[SEQ * DIM + DIM:], _row(bn1e), wn2e, _row(bn2e),
        _row(gn_e), _row(bn_e))

    sp1 = params["steps"][0]
    (ws1, bs1), (ws2, bs2) = sp1["edge"]
    gs1, bsl1 = sp1["edge_ln"]
    (wn1s, bn1s), (wn2s, bn2s) = sp1["node"]
    gn1, bnl1 = sp1["node_ln"]

    e1h, agg1p = [], []
    for idx3d, dst3d, pdst, ne in halves:
        G1 = _sc_gather(xp0, idx3d, 2 * ne, 72)
        e1_, eu1_ = _edge_step1(
            G1, pdst, we1[:DIM], _row(we1[DIM]), _row(be1), we2, _row(be2),
            _row(ge_e), _row(be_l),
            ws1[:LATENT], ws1[LATENT:2 * LATENT], ws1[2 * LATENT:],
            _row(bs1), ws2, _row(bs2), _row(gs1), _row(bsl1))
        e1h.append(e1_)
        agg1p.append(_sc_scatter_add(eu1_, dst3d, ne))
    agg1 = sum(agg1p[1:], agg1p[0]).reshape(2, 2 * PROWS, LATENT)
    x1, x1t = _node_update(x0, agg1, wn1s[:LATENT], wn1s[LATENT:], _row(bn1s),
                           wn2s, _row(bn2s), _row(gn1), _row(bnl1))

    sp2 = params["steps"][1]
    (ws1b, bs1b), (ws2b, bs2b) = sp2["edge"]
    gs2, bsl2 = sp2["edge_ln"]
    (wn1t, bn1t), (wn2t, bn2t) = sp2["node"]
    gn2, bnl2 = sp2["node_ln"]

    agg2p = []
    for h, (idx3d, dst3d, pdst, ne) in enumerate(halves):
        G2 = _sc_gather(x1t, idx3d, 2 * ne, LATENT)
        eu2_ = _edge_step2(
            e1h[h], G2, pdst, ws1b[:LATENT], ws1b[LATENT:2 * LATENT],
            ws1b[2 * LATENT:], _row(bs1b), ws2b, _row(bs2b), _row(gs2),
            _row(bsl2))
        agg2p.append(_sc_scatter_add(eu2_, dst3d, ne))
    agg2 = sum(agg2p[1:], agg2p[0]).reshape(2, 2 * PROWS, LATENT)

    return _node_update_decode(
        x1, agg2, wn1t[:LATENT], wn1t[LATENT:], _row(bn1t), wn2t, _row(bn2t),
        _row(gn2), _row(bnl2), wd1, _row(bd1), wd2, _row(bd2))


# speedup vs baseline: 2.4655x; 2.4655x over previous
import jax
import jax.numpy as jnp
from jax import lax
from jax.experimental import pallas as pl
from jax.experimental.pallas import tpu as pltpu
from jax.experimental.pallas import tpu_sc as plsc

N = 50000
E = 800000
DIM = 2
SEQ = 5
LATENT = 64
H = 0.05
CLAMP = 1.0
VEL_W = 1.0
STD = 1.0
MEAN = 0.0
B_LO = 0.0
B_HI = 1.0

NC = 2
NS = 16
NW = NC * NS
SUB = 80
NSUB = 8
CH = SUB * NSUB

HALF = N // 2
PROWS = 12544
TRASH = PROWS - 1

BN = 5000
BE = 4000

_f32 = jnp.float32


def _ln(h, g, b):
    m = jnp.mean(h, axis=-1, keepdims=True)
    v = jnp.mean((h - m) * (h - m), axis=-1, keepdims=True)
    return (h - m) * lax.rsqrt(v + 1e-5) * g + b


def _sc_gather(table, idx3d, K, outw):
    nch = K // CH
    niter = -(-nch // NW)
    mesh = plsc.VectorSubcoreMesh(core_axis_name="c", subcore_axis_name="s")

    def body(table_hbm, idx_hbm, out_hbm, idx_v, rows_v, sem, semw):
        cid = lax.axis_index("c")
        sid = lax.axis_index("s")
        wid = sid * NC + cid

        def _drain_put(ch, j):
            pltpu.make_async_copy(
                rows_v.at[pl.ds(j * SUB, SUB)],
                out_hbm.at[pl.ds(ch * CH + j * SUB, SUB)],
                semw,
            ).wait()

        def it(i, carry):
            ch = i * NW + wid

            @pl.when(ch < nch)
            def _do():
                pltpu.sync_copy(idx_hbm.at[ch], idx_v)
                gets = []
                for j in range(NSUB):
                    @pl.when(i > 0)
                    def _(_j=j):
                        _drain_put(ch, _j)

                    gets.append(pltpu.async_copy(
                        table_hbm.at[idx_v.at[j]],
                        rows_v.at[pl.ds(j * SUB, SUB)],
                        sem,
                    ))
                for j in range(NSUB):
                    gets[j].wait()
                    pltpu.async_copy(
                        rows_v.at[pl.ds(j * SUB, SUB)],
                        out_hbm.at[pl.ds(ch * CH + j * SUB, SUB)],
                        semw,
                    )

            return carry

        lax.fori_loop(0, niter, it, 0)
        for j in range(NSUB):
            _drain_put(0, j)

    run = pl.kernel(
        body,
        out_type=jax.ShapeDtypeStruct((K, 128), _f32),
        mesh=mesh,
        scratch_types=[
            pltpu.VMEM((NSUB, SUB), jnp.int32),
            pltpu.VMEM((CH, 128), _f32),
            pltpu.SemaphoreType.DMA,
            pltpu.SemaphoreType.DMA,
        ],
    )
    return run(table, idx3d)


def _sc_scatter_add(eupd_sl, dst3d, ne):
    TS = PROWS // NS
    nch = ne // CH
    niter = -(-nch // NS)
    mesh = plsc.VectorSubcoreMesh(core_axis_name="c", subcore_axis_name="s")

    def body(eupd_hbm, dst_hbm, zeros_hbm, out_hbm,
             dst_a, dst_b, lidx_v, rows_a, rows_b, zero_v, shared,
             seml, sema, semd):
        cid = lax.axis_index("c")
        sid = lax.axis_index("s")
        node0 = cid * HALF
        bufs = (rows_a, rows_b)
        dbufs = (dst_a, dst_b)

        pltpu.sync_copy(zeros_hbm, zero_v)
        zcopies = [
            pltpu.async_copy(zero_v if k < 16 else zero_v.at[pl.ds(0, 16)],
                             shared.at[pl.ds(sid * TS + k * 48,
                                             48 if k < 16 else 16)],
                             sema)
            for k in range(17)
        ]
        for z in zcopies:
            z.wait()
        plsc.subcore_barrier()

        def chunk(ch, dbuf, nbuf, next_ch):
            @pl.when(next_ch < nch)
            def _():
                pltpu.async_copy(dst_hbm.at[next_ch], nbuf, semd)

            for j in range(NSUB):
                for k in range(SUB // 16):
                    d = dbuf[j, pl.ds(k * 16, 16)]
                    li = d - node0
                    ok = (li >= 0) & (li < HALF)
                    r = lax.shift_right_logical(li, 1)
                    lidx_v[j, pl.ds(k * 16, 16)] = jnp.where(ok, r, TRASH)
            loads = [None] * NSUB
            adds = [None] * NSUB
            loads[0] = pltpu.async_copy(
                eupd_hbm.at[pl.ds(ch * CH, SUB)], bufs[0], seml)
            for sb in range(NSUB):
                pr = sb & 1
                loads[sb].wait()
                adds[sb] = pltpu.async_copy(
                    bufs[pr], shared.at[lidx_v.at[sb]], sema, add=True)
                if sb + 1 < NSUB:
                    if sb >= 1:
                        adds[sb - 1].wait()
                    loads[sb + 1] = pltpu.async_copy(
                        eupd_hbm.at[pl.ds(ch * CH + (sb + 1) * SUB, SUB)],
                        bufs[1 - pr], seml)
            adds[NSUB - 2].wait()
            adds[NSUB - 1].wait()

        def _drain_dst(nbuf, next_ch):
            pltpu.make_async_copy(dst_hbm.at[next_ch], nbuf, semd).wait()

        pltpu.sync_copy(dst_hbm.at[sid], dst_a)

        def it(m, carry):
            ch0 = (2 * m) * NS + sid
            ch1 = ch0 + NS
            ch2 = ch0 + 2 * NS

            @pl.when(ch0 < nch)
            def _c0():
                chunk(ch0, dst_a, dst_b, ch1)

            @pl.when(ch1 < nch)
            def _c1():
                _drain_dst(dst_b, ch1)
                chunk(ch1, dst_b, dst_a, ch2)

            @pl.when(ch2 < nch)
            def _c2():
                _drain_dst(dst_a, ch2)

            return carry

        lax.fori_loop(0, -(-niter // 2), it, 0)
        plsc.subcore_barrier()

        puts = []
        for k in range(10):
            r0 = sid * TS + k * 80
            nrow = 80 if k < 9 else 784 - 720
            buf = bufs[k & 1]
            if k >= 2:
                puts[k - 2].wait()
            pltpu.sync_copy(shared.at[pl.ds(r0, nrow)],
                            buf.at[pl.ds(0, nrow)])
            puts.append(pltpu.async_copy(
                buf.at[pl.ds(0, nrow)],
                out_hbm.at[pl.ds(cid * PROWS + r0, nrow)], seml))
        puts[-2].wait()
        puts[-1].wait()

    run = pl.kernel(
        body,
        out_type=jax.ShapeDtypeStruct((2 * PROWS, 128), _f32),
        mesh=mesh,
        scratch_types=[
            pltpu.VMEM((NSUB, SUB), jnp.int32),
            pltpu.VMEM((NSUB, SUB), jnp.int32),
            pltpu.VMEM((NSUB, SUB), jnp.int32),
            pltpu.VMEM((SUB, 128), _f32),
            pltpu.VMEM((SUB, 128), _f32),
            pltpu.VMEM((48, 128), _f32),
            pltpu.VMEM_SHARED((PROWS, 128), _f32),
            pltpu.SemaphoreType.DMA,
            pltpu.SemaphoreType.DMA,
            pltpu.SemaphoreType.DMA,
        ],
    )
    zeros = jnp.zeros((48, 128), _f32)
    return run(eupd_sl, dst3d, zeros)


def _full(shape):
    return pl.BlockSpec(shape, lambda i: (0, 0))


_AGG_SPEC = pl.BlockSpec((1, BN, LATENT), lambda i: (i // 5, i % 5, 0))


def _node_encode(pos, vel, w1v, w1dl, w1du, b1, w2, b2, g, b):
    def body(pos_ref, vel_ref, w1v_r, w1dl_r, w1du_r, b1_r, w2_r, b2_r,
             g_r, b_r, x_ref, xp_ref):
        p = pos_ref[...]
        v = vel_ref[...] * VEL_W
        dl = jnp.clip(jnp.abs(p - B_LO) / H, -CLAMP, CLAMP)
        du = jnp.clip(jnp.abs(B_HI - p) / H, -CLAMP, CLAMP)
        h = jnp.maximum(
            jnp.dot(v, w1v_r[...], preferred_element_type=_f32)
            + jnp.dot(dl, w1dl_r[...], preferred_element_type=_f32)
            + jnp.dot(du, w1du_r[...], preferred_element_type=_f32)
            + b1_r[...], 0.0)
        h2 = jnp.dot(h, w2_r[...], preferred_element_type=_f32) + b2_r[...]
        x = _ln(h2, g_r[...], b_r[...])
        x_ref[...] = x
        xp_ref[...] = jnp.concatenate(
            [x, p, jnp.zeros((p.shape[0], 128 - LATENT - DIM), _f32)], axis=1)

    return pl.pallas_call(
        body,
        grid=(N // BN,),
        in_specs=[
            pl.BlockSpec((BN, DIM), lambda i: (i, 0)),
            pl.BlockSpec((BN, SEQ * DIM), lambda i: (i, 0)),
            _full((SEQ * DIM, LATENT)), _full((DIM, LATENT)),
            _full((DIM, LATENT)), _full((1, LATENT)),
            _full((LATENT, LATENT)), _full((1, LATENT)),
            _full((1, LATENT)), _full((1, LATENT)),
        ],
        out_specs=[
            pl.BlockSpec((BN, LATENT), lambda i: (i, 0)),
            pl.BlockSpec((BN, 128), lambda i: (i, 0)),
        ],
        out_shape=[
            jax.ShapeDtypeStruct((N, LATENT), _f32),
            jax.ShapeDtypeStruct((N, 128), _f32),
        ],
    )(pos, vel, w1v, w1dl, w1du, b1, w2, b2, g, b)


def _edge_step1(G1, pdst, wr, wd, be1, we2, be2, ge, bel,
                wse, wss, wsr, bs1, ws2, bs2, gs, bsl):
    def body(s_ref, r_ref, p_ref, wr_r, wd_r, be1_r, we2_r, be2_r, ge_r,
             bel_r, wse_r, wss_r, wsr_r, bs1_r, ws2_r, bs2_r, gs_r, bsl_r,
             e1_ref, eu_ref):
        s = s_ref[...]
        r = r_ref[...]
        sx = s[:, :LATENT]
        rx = r[:, :LATENT]
        rel = (s[:, LATENT:LATENT + DIM] - r[:, LATENT:LATENT + DIM]) / H
        rd = jnp.sqrt(jnp.sum(rel * rel, axis=-1, keepdims=True))
        he = jnp.maximum(
            jnp.dot(rel, wr_r[...], preferred_element_type=_f32)
            + rd * wd_r[...] + be1_r[...], 0.0)
        e = _ln(jnp.dot(he, we2_r[...], preferred_element_type=_f32)
                + be2_r[...], ge_r[...], bel_r[...])
        h = jnp.maximum(
            jnp.dot(e, wse_r[...], preferred_element_type=_f32)
            + jnp.dot(sx, wss_r[...], preferred_element_type=_f32)
            + jnp.dot(rx, wsr_r[...], preferred_element_type=_f32)
            + bs1_r[...], 0.0)
        eu = _ln(jnp.dot(h, ws2_r[...], preferred_element_type=_f32)
                 + bs2_r[...], gs_r[...], bsl_r[...])
        e1_ref[...] = e + eu
        p = p_ref[...]
        eu_ref[...] = jnp.concatenate([eu * (1.0 - p), eu * p], axis=1)

    ne = pdst.shape[0]
    nb = ne // BE
    return pl.pallas_call(
        body,
        grid=(nb,),
        in_specs=[
            pl.BlockSpec((BE, 128), lambda i: (i, 0)),
            pl.BlockSpec((BE, 128), lambda i, _nb=nb: (_nb + i, 0)),
            pl.BlockSpec((BE, 1), lambda i: (i, 0)),
            _full((DIM, LATENT)), _full((1, LATENT)), _full((1, LATENT)),
            _full((LATENT, LATENT)), _full((1, LATENT)),
            _full((1, LATENT)), _full((1, LATENT)),
            _full((LATENT, LATENT)), _full((LATENT, LATENT)),
            _full((LATENT, LATENT)), _full((1, LATENT)),
            _full((LATENT, LATENT)), _full((1, LATENT)),
            _full((1, LATENT)), _full((1, LATENT)),
        ],
        out_specs=[
            pl.BlockSpec((BE, LATENT), lambda i: (i, 0)),
            pl.BlockSpec((BE, 128), lambda i: (i, 0)),
        ],
        out_shape=[
            jax.ShapeDtypeStruct((ne, LATENT), _f32),
            jax.ShapeDtypeStruct((ne, 128), _f32),
        ],
    )(G1, G1, pdst, wr, wd, be1, we2, be2, ge, bel,
      wse, wss, wsr, bs1, ws2, bs2, gs, bsl)


def _edge_step2(e1, G2, pdst, wse, wss, wsr, bs1, ws2, bs2, gs, bsl):
    def body(e_ref, s_ref, r_ref, p_ref, wse_r, wss_r, wsr_r, bs1_r, ws2_r,
             bs2_r, gs_r, bsl_r, eu_ref):
        h = jnp.maximum(
            jnp.dot(e_ref[...], wse_r[...], preferred_element_type=_f32)
            + jnp.dot(s_ref[...][:, :LATENT], wss_r[...],
                      preferred_element_type=_f32)
            + jnp.dot(r_ref[...][:, :LATENT], wsr_r[...],
                      preferred_element_type=_f32)
            + bs1_r[...], 0.0)
        eu = _ln(jnp.dot(h, ws2_r[...], preferred_element_type=_f32)
                 + bs2_r[...], gs_r[...], bsl_r[...])
        p = p_ref[...]
        eu_ref[...] = jnp.concatenate([eu * (1.0 - p), eu * p], axis=1)

    ne = pdst.shape[0]
    nb = ne // BE
    return pl.pallas_call(
        body,
        grid=(nb,),
        in_specs=[
            pl.BlockSpec((BE, LATENT), lambda i: (i, 0)),
            pl.BlockSpec((BE, 128), lambda i: (i, 0)),
            pl.BlockSpec((BE, 128), lambda i, _nb=nb: (_nb + i, 0)),
            pl.BlockSpec((BE, 1), lambda i: (i, 0)),
            _full((LATENT, LATENT)), _full((LATENT, LATENT)),
            _full((LATENT, LATENT)), _full((1, LATENT)),
            _full((LATENT, LATENT)), _full((1, LATENT)),
            _full((1, LATENT)), _full((1, LATENT)),
        ],
        out_specs=pl.BlockSpec((BE, 128), lambda i: (i, 0)),
        out_shape=jax.ShapeDtypeStruct((ne, 128), _f32),
    )(e1, G2, G2, pdst, wse, wss, wsr, bs1, ws2, bs2, gs, bsl)


def _node_update(x, agg, wnx, wna, bn1, wn2, bn2, gn, bnl):
    def body(x_ref, a_ref, wnx_r, wna_r, bn1_r, wn2_r, bn2_r, gn_r, bnl_r,
             o_ref, ot_ref):
        x_ = x_ref[...]
        h = jnp.maximum(
            jnp.dot(x_, wnx_r[...], preferred_element_type=_f32)
            + jnp.dot(a_ref[0], wna_r[...], preferred_element_type=_f32)
            + bn1_r[...], 0.0)
        x1 = x_ + _ln(
            jnp.dot(h, wn2_r[...], preferred_element_type=_f32) + bn2_r[...],
            gn_r[...], bnl_r[...])
        o_ref[...] = x1
        ot_ref[...] = jnp.concatenate(
            [x1, jnp.zeros((x1.shape[0], 128 - LATENT), _f32)], axis=1)

    return pl.pallas_call(
        body,
        grid=(N // BN,),
        in_specs=[
            pl.BlockSpec((BN, LATENT), lambda i: (i, 0)),
            _AGG_SPEC,
            _full((LATENT, LATENT)), _full((LATENT, LATENT)),
            _full((1, LATENT)), _full((LATENT, LATENT)), _full((1, LATENT)),
            _full((1, LATENT)), _full((1, LATENT)),
        ],
        out_specs=[
            pl.BlockSpec((BN, LATENT), lambda i: (i, 0)),
            pl.BlockSpec((BN, 128), lambda i: (i, 0)),
        ],
        out_shape=[
            jax.ShapeDtypeStruct((N, LATENT), _f32),
            jax.ShapeDtypeStruct((N, 128), _f32),
        ],
    )(x, agg, wnx, wna, bn1, wn2, bn2, gn, bnl)


def _node_update_decode(x, agg, wnx, wna, bn1, wn2, bn2, gn, bnl,
                        wd1, bd1, wd2, bd2):
    def body(x_ref, a_ref, wnx_r, wna_r, bn1_r, wn2_r, bn2_r, gn_r, bnl_r,
             wd1_r, bd1_r, wd2_r, bd2_r, o_ref):
        x_ = x_ref[...]
        h = jnp.maximum(
            jnp.dot(x_, wnx_r[...], preferred_element_type=_f32)
            + jnp.dot(a_ref[0], wna_r[...], preferred_element_type=_f32)
            + bn1_r[...], 0.0)
        x2 = x_ + _ln(
            jnp.dot(h, wn2_r[...], preferred_element_type=_f32) + bn2_r[...],
            gn_r[...], bnl_r[...])
        hd = jnp.maximum(
            jnp.dot(x2, wd1_r[...], preferred_element_type=_f32) + bd1_r[...],
            0.0)
        o_ref[...] = (jnp.dot(hd, wd2_r[...], preferred_element_type=_f32)
                      + bd2_r[...]) * STD + MEAN

    return pl.pallas_call(
        body,
        grid=(N // BN,),
        in_specs=[
            pl.BlockSpec((BN, LATENT), lambda i: (i, 0)),
            _AGG_SPEC,
            _full((LATENT, LATENT)), _full((LATENT, LATENT)),
            _full((1, LATENT)), _full((LATENT, LATENT)), _full((1, LATENT)),
            _full((1, LATENT)), _full((1, LATENT)),
            _full((LATENT, LATENT)), _full((1, LATENT)),
            _full((LATENT, DIM)), _full((1, DIM)),
        ],
        out_specs=pl.BlockSpec((BN, DIM), lambda i: (i, 0)),
        out_shape=jax.ShapeDtypeStruct((N, DIM), _f32),
    )(x, agg, wnx, wna, bn1, wn2, bn2, gn, bnl, wd1, bd1, wd2, bd2)


def _row(v):
    return v.reshape(1, -1)


def kernel(current_position, velocity_sequence, edge_index, params):
    pos = current_position.astype(_f32)
    vel = velocity_sequence.astype(_f32).reshape(N, SEQ * DIM)
    ei = edge_index.astype(jnp.int32)
    PARTS = (272000, 272000, 256000)
    halves = []
    off = 0
    for ne in PARTS:
        eh = lax.slice(ei, (0, off), (2, off + ne))
        off += ne
        halves.append((
            eh.reshape(2 * ne // CH, NSUB, SUB),
            eh[1].reshape(ne // CH, NSUB, SUB),
            (eh[1] & 1).astype(_f32).reshape(ne, 1),
            ne,
        ))

    (wn1, bn1e), (wn2e, bn2e) = params["node_enc"]
    gn_e, bn_e = params["node_enc_ln"]
    (we1, be1), (we2, be2) = params["edge_enc"]
    ge_e, be_l = params["edge_enc_ln"]
    (wd1, bd1), (wd2, bd2) = params["dec"]

    x0, xp0 = _node_encode(
        pos, vel,
        wn1[:SEQ * DIM], wn1[SEQ * DIM:SEQ * DIM + DIM],
        wn1[SEQ * DIM + DIM:], _row(bn1e), wn2e, _row(bn2e),
        _row(gn_e), _row(bn_e))

    sp1 = params["steps"][0]
    (ws1, bs1), (ws2, bs2) = sp1["edge"]
    gs1, bsl1 = sp1["edge_ln"]
    (wn1s, bn1s), (wn2s, bn2s) = sp1["node"]
    gn1, bnl1 = sp1["node_ln"]

    e1h, agg1p = [], []
    for idx3d, dst3d, pdst, ne in halves:
        G1 = _sc_gather(xp0, idx3d, 2 * ne, 72)
        e1_, eu1_ = _edge_step1(
            G1, pdst, we1[:DIM], _row(we1[DIM]), _row(be1), we2, _row(be2),
            _row(ge_e), _row(be_l),
            ws1[:LATENT], ws1[LATENT:2 * LATENT], ws1[2 * LATENT:],
            _row(bs1), ws2, _row(bs2), _row(gs1), _row(bsl1))
        e1h.append(e1_)
        agg1p.append(_sc_scatter_add(eu1_, dst3d, ne))
    agg1 = sum(agg1p[1:], agg1p[0]).reshape(2, 2 * PROWS, LATENT)
    x1, x1t = _node_update(x0, agg1, wn1s[:LATENT], wn1s[LATENT:], _row(bn1s),
                           wn2s, _row(bn2s), _row(gn1), _row(bnl1))

    sp2 = params["steps"][1]
    (ws1b, bs1b), (ws2b, bs2b) = sp2["edge"]
    gs2, bsl2 = sp2["edge_ln"]
    (wn1t, bn1t), (wn2t, bn2t) = sp2["node"]
    gn2, bnl2 = sp2["node_ln"]

    agg2p = []
    for h, (idx3d, dst3d, pdst, ne) in enumerate(halves):
        G2 = _sc_gather(x1t, idx3d, 2 * ne, LATENT)
        eu2_ = _edge_step2(
            e1h[h], G2, pdst, ws1b[:LATENT], ws1b[LATENT:2 * LATENT],
            ws1b[2 * LATENT:], _row(bs1b), ws2b, _row(bs2b), _row(gs2),
            _row(bsl2))
        agg2p.append(_sc_scatter_add(eu2_, dst3d, ne))
    agg2 = sum(agg2p[1:], agg2p[0]).reshape(2, 2 * PROWS, LATENT)

    return _node_update_decode(
        x1, agg2, wn1t[:LATENT], wn1t[LATENT:], _row(bn1t), wn2t, _row(bn2t),
        _row(gn2), _row(bnl2), wd1, _row(bd1), wd2, _row(bd2))
